# Initial kernel scaffold; baseline (speedup 1.0000x reference)
import jax
import jax.numpy as jnp
from jax.experimental import pallas as pl


def kernel(prefix, table):
    raise NotImplementedError("write your pallas kernel here")


# speedup vs baseline: 2.3826x; 2.3826x over previous
import functools

import jax
import jax.numpy as jnp
from jax import lax
from jax.experimental import pallas as pl
from jax.experimental.pallas import tpu as pltpu
from jax.experimental.pallas import tpu_sc as plsc

_NC = 2
_NS = 16
_NW = _NC * _NS


def _sc_gather(idx3, table):
    n_steps, k = idx3.shape[1], idx3.shape[2]
    b_per_w = n_steps * k
    n_idx = _NW * b_per_w
    emb = table.shape[1]
    mesh = plsc.VectorSubcoreMesh(core_axis_name="c", subcore_axis_name="s")

    @functools.partial(
        pl.kernel,
        mesh=mesh,
        out_type=jax.ShapeDtypeStruct((n_idx, emb), jnp.float32),
        scratch_types=[
            pltpu.VMEM((n_steps, k), jnp.int32),
            pltpu.VMEM((k, emb), jnp.float32),
            pltpu.SemaphoreType.DMA,
        ],
    )
    def kern(idx_hbm, table_hbm, out_hbm, idx_v, rows_v, sem):
        wid = lax.axis_index("s") * _NC + lax.axis_index("c")
        base = wid * b_per_w
        pltpu.sync_copy(idx_hbm.at[wid], idx_v)

        def body(i, carry):
            pltpu.async_copy(table_hbm.at[idx_v.at[i]], rows_v, sem).wait()
            pltpu.sync_copy(rows_v, out_hbm.at[pl.ds(base + i * k, k)])
            return carry

        lax.fori_loop(0, n_steps, body, 0)

    return kern(idx3, table)


def kernel(prefix, table):
    b, s = prefix.shape
    k = 2
    idx3 = prefix.reshape(_NW, (b * s) // (_NW * k), k).astype(jnp.int32)
    out = _sc_gather(idx3, table)
    return out.reshape(b, s, table.shape[1])


# speedup vs baseline: 4.5307x; 1.9016x over previous
import functools

import jax
import jax.numpy as jnp
from jax import lax
from jax.experimental import pallas as pl
from jax.experimental.pallas import tpu as pltpu
from jax.experimental.pallas import tpu_sc as plsc

_NC = 2
_NS = 16
_NW = _NC * _NS


def _sc_scatter_rows(sidx2, ord2, table):
    b_per_w = sidx2.shape[1]
    emb = table.shape[1]
    n_idx = _NW * b_per_w
    mesh = plsc.VectorSubcoreMesh(core_axis_name="c", subcore_axis_name="s")

    @functools.partial(
        pl.kernel,
        mesh=mesh,
        out_type=jax.ShapeDtypeStruct((n_idx, emb), jnp.float32),
        scratch_types=[
            pltpu.VMEM((b_per_w,), jnp.int32),
            pltpu.VMEM((b_per_w,), jnp.int32),
            pltpu.VMEM((1, emb), jnp.float32),
        ],
    )
    def kern(sidx_hbm, ord_hbm, table_hbm, out_hbm, idx_v, ord_v, row_v):
        wid = lax.axis_index("s") * _NC + lax.axis_index("c")
        pltpu.sync_copy(sidx_hbm.at[wid], idx_v)
        pltpu.sync_copy(ord_hbm.at[wid], ord_v)

        def body(j, prev):
            vec_i = idx_v[pl.ds(j * 16, 16)]
            vec_p = ord_v[pl.ds(j * 16, 16)]
            for l in range(16):
                v = vec_i[l]
                p = vec_p[l]

                @pl.when(v != prev)
                def _(v=v):
                    pltpu.sync_copy(table_hbm.at[pl.ds(v, 1)], row_v)

                pltpu.sync_copy(row_v, out_hbm.at[pl.ds(p, 1)])
                prev = v
            return prev

        lax.fori_loop(0, b_per_w // 16, body, jnp.int32(-1))

    return kern(sidx2, ord2, table)


def kernel(prefix, table):
    b, s = prefix.shape
    idxf = prefix.reshape(-1).astype(jnp.int32)
    order = jnp.argsort(idxf).astype(jnp.int32)
    sidx = jnp.take(idxf, order)
    out = _sc_scatter_rows(sidx.reshape(_NW, -1), order.reshape(_NW, -1), table)
    return out.reshape(b, s, table.shape[1])
